# Initial kernel scaffold; baseline (speedup 1.0000x reference)
import jax
import jax.numpy as jnp
from jax.experimental import pallas as pl


def kernel(input, weight):
    raise NotImplementedError("write your pallas kernel here")


# speedup vs baseline: 1.3478x; 1.3478x over previous
import jax
import jax.numpy as jnp
from jax.experimental import pallas as pl
from jax.experimental.pallas import tpu as pltpu
from jax.experimental.pallas import tpu_sc as plsc

_WINDOW = 128


def _gather_kernel(num_indices, dim, dtype):
    mesh = plsc.VectorSubcoreMesh(core_axis_name="core",
                                  subcore_axis_name="subcore")

    @pl.kernel(out_type=jax.ShapeDtypeStruct((num_indices, dim), dtype),
               mesh=mesh,
               compiler_params=pltpu.CompilerParams(use_tc_tiling_on_sc=False))
    def gather(w_hbm, i_hbm, o_hbm):
        def body(i_vmem, o_vmem):
            pltpu.sync_copy(w_hbm.at[i_vmem.at[0]], o_vmem)

        pltpu.emit_pipeline(
            body,
            grid=(num_indices // _WINDOW,),
            in_specs=[pl.BlockSpec((1, _WINDOW), index_map=lambda i: (0, i))],
            out_specs=[pl.BlockSpec((_WINDOW, dim), index_map=lambda i: (i, 0))],
            core_axis_name=("core", "subcore"),
            dimension_semantics=(pltpu.PARALLEL,),
        )(i_hbm, o_hbm)

    return gather


def kernel(input, weight):
    b, t = input.shape
    n = b * t
    dim = weight.shape[1]
    idx = input.reshape(1, n).astype(jnp.int32)
    out = _gather_kernel(n, dim, weight.dtype)(weight, idx)
    return out.reshape(b, t, dim)


# speedup vs baseline: 1.4931x; 1.1078x over previous
import jax
import jax.numpy as jnp
from jax.experimental import pallas as pl
from jax.experimental.pallas import tpu as pltpu
from jax.experimental.pallas import tpu_sc as plsc

_WINDOW = 128
_K = 8


def _gather_kernel(num_indices, dim, dtype):
    mesh = plsc.VectorSubcoreMesh(core_axis_name="core",
                                  subcore_axis_name="subcore")

    @pl.kernel(out_type=jax.ShapeDtypeStruct((num_indices, dim), dtype),
               mesh=mesh,
               scratch_types=[pltpu.SemaphoreType.DMA],
               compiler_params=pltpu.CompilerParams(use_tc_tiling_on_sc=False))
    def gather(w_hbm, i_hbm, o_hbm, sem):
        def body(i_vmem, o_vmem):
            copies = [
                pltpu.async_copy(w_hbm.at[i_vmem.at[j]],
                                 o_vmem.at[pl.ds(j * _WINDOW, _WINDOW)], sem)
                for j in range(_K)
            ]
            for c in copies:
                c.wait()

        pltpu.emit_pipeline(
            body,
            grid=(num_indices // (_K * _WINDOW),),
            in_specs=[pl.BlockSpec((_K, _WINDOW), index_map=lambda i: (i, 0))],
            out_specs=[pl.BlockSpec((_K * _WINDOW, dim),
                                    index_map=lambda i: (i, 0))],
            core_axis_name=("core", "subcore"),
            dimension_semantics=(pltpu.PARALLEL,),
        )(i_hbm, o_hbm)

    return gather


def kernel(input, weight):
    b, t = input.shape
    n = b * t
    dim = weight.shape[1]
    idx = input.reshape(n // _WINDOW, _WINDOW).astype(jnp.int32)
    out = _gather_kernel(n, dim, weight.dtype)(weight, idx)
    return out.reshape(b, t, dim)


# speedup vs baseline: 1.6012x; 1.0724x over previous
import jax
import jax.numpy as jnp
from jax.experimental import pallas as pl
from jax.experimental.pallas import tpu as pltpu
from jax.experimental.pallas import tpu_sc as plsc

_WINDOW = 128
_K = 8
_U = 16


def _repack_table(wt):
    n = wt.shape[1]
    grid = (n + 128 * _U - 1) // (128 * _U)

    def body(x_ref, o_ref):
        for u in range(_U):
            o_ref[pl.ds(128 * (u // 4), 128), pl.ds(32 * (u % 4), 32)] = (
                x_ref[:, pl.ds(128 * u, 128)].T)

    return pl.pallas_call(
        body,
        grid=(grid,),
        in_specs=[pl.BlockSpec((32, 128 * _U), lambda g: (0, g))],
        out_specs=pl.BlockSpec((128 * _U // 4, 128), lambda g: (g, 0)),
        out_shape=jax.ShapeDtypeStruct((grid * 128 * _U // 4, 128), wt.dtype),
    )(wt)


def _permute_indices(i):
    m = i >> 7
    r = i & 127
    g = m >> 4
    u = m & 15
    return 2048 * g + 512 * (u >> 2) + 4 * r + (u & 3)


def _gather_rows(table, idx, num_indices, dim):
    mesh = plsc.VectorSubcoreMesh(core_axis_name="core",
                                  subcore_axis_name="subcore")

    @pl.kernel(out_type=jax.ShapeDtypeStruct((num_indices, dim), jnp.float32),
               mesh=mesh,
               scratch_types=[pltpu.SemaphoreType.DMA],
               compiler_params=pltpu.CompilerParams(use_tc_tiling_on_sc=False))
    def gather(w_hbm, i_hbm, o_hbm, sem):
        def body(i_vmem, o_vmem):
            copies = [
                pltpu.async_copy(w_hbm.at[i_vmem.at[j]],
                                 o_vmem.at[pl.ds(j * _WINDOW, _WINDOW)], sem)
                for j in range(_K)
            ]
            for c in copies:
                c.wait()

        pltpu.emit_pipeline(
            body,
            grid=(num_indices // (_K * _WINDOW),),
            in_specs=[pl.BlockSpec((_K, _WINDOW), index_map=lambda i: (i, 0))],
            out_specs=[pl.BlockSpec((_K * _WINDOW, dim),
                                    index_map=lambda i: (i, 0))],
            core_axis_name=("core", "subcore"),
            dimension_semantics=(pltpu.PARALLEL,),
        )(i_hbm, o_hbm)

    return gather(table, idx)


def kernel(input, weight):
    b_len, t_len = input.shape
    n = b_len * t_len
    dim = weight.shape[1]

    w4 = _repack_table(weight.T)
    table = w4.reshape(w4.shape[0] * 4, dim)

    idx = _permute_indices(input).reshape(n // _WINDOW, _WINDOW)
    rows = _gather_rows(table, idx, n, dim)
    return rows.reshape(b_len, t_len, dim)
